# baseline (speedup 1.0000x reference)
import jax
import jax.numpy as jnp
from jax import lax
from jax.experimental import pallas as pl
from jax.experimental.pallas import tpu as pltpu
from jax.experimental.pallas import tpu_sc as plsc

_D = 64
_V = 49
_BATCH = 16384
_HIST = 200
_NTOT = _BATCH * _HIST
_NC = 2
_NS = 16
_NW = _NC * _NS
_PER_W = _NTOT // _NW
_CHUNK = 800
_NCH = _PER_W // _CHUNK
_G = _CHUNK // 16


def _body(idx_hbm, table_hbm, out_hbm, table_v, idx_v0, idx_v1, rows_v0,
          rows_v1, idx_sem, out_sem):
    idx_bufs = (idx_v0, idx_v1)
    row_bufs = (rows_v0, rows_v1)
    wid = lax.axis_index("s") * _NC + lax.axis_index("c")
    base = wid * _PER_W
    pltpu.sync_copy(table_hbm, table_v)

    for b in range(2):
        off = base + b * _CHUNK
        pltpu.async_copy(idx_hbm.at[pl.ds(off, _CHUNK)], idx_bufs[b],
                         idx_sem.at[b])

    iota = lax.iota(jnp.int32, 16)

    def compute(b):
        @plsc.parallel_loop(0, _G, step=1, unroll=2)
        def g_step(g):
            idx16 = idx_bufs[b][pl.ds(g * 16, 16)]
            src0 = idx16 * _D
            dst0 = (g * 16 + iota) * _D
            for j0 in range(0, _D, 8):
                xs = [plsc.load_gather(table_v, [src0 + (j0 + t)])
                      for t in range(8)]
                for t in range(8):
                    plsc.store_scatter(row_bufs[b], [dst0 + (j0 + t)], xs[t])

    def pair_step(i2, carry):
        for b in range(2):
            i = i2 * 2 + b
            off = base + i * _CHUNK
            ooff = off * _D
            pltpu.make_async_copy(idx_hbm.at[pl.ds(off, _CHUNK)],
                                  idx_bufs[b], idx_sem.at[b]).wait()

            @pl.when(i2 >= 1)
            def _wait_out():
                pltpu.make_async_copy(row_bufs[b],
                                      out_hbm.at[pl.ds(ooff, _CHUNK * _D)],
                                      out_sem.at[b]).wait()

            compute(b)
            pltpu.async_copy(row_bufs[b],
                             out_hbm.at[pl.ds(ooff, _CHUNK * _D)],
                             out_sem.at[b])

            @pl.when(i2 < _NCH // 2 - 1)
            def _prefetch_idx():
                off2 = base + (i + 2) * _CHUNK
                pltpu.async_copy(idx_hbm.at[pl.ds(off2, _CHUNK)],
                                 idx_bufs[b], idx_sem.at[b])
        return carry

    lax.fori_loop(0, _NCH // 2, pair_step, 0)

    for b in range(2):
        pltpu.make_async_copy(row_bufs[b],
                              out_hbm.at[pl.ds(base * _D, _CHUNK * _D)],
                              out_sem.at[b]).wait()


_mesh = plsc.VectorSubcoreMesh(core_axis_name="c", subcore_axis_name="s")

_gather = pl.kernel(
    _body,
    out_type=jax.ShapeDtypeStruct((_NTOT * _D,), jnp.float32),
    mesh=_mesh,
    compiler_params=pltpu.CompilerParams(needs_layout_passes=False),
    scratch_types=[
        pltpu.VMEM((_V * _D,), jnp.float32),
        pltpu.VMEM((_CHUNK,), jnp.int32),
        pltpu.VMEM((_CHUNK,), jnp.int32),
        pltpu.VMEM((_CHUNK * _D,), jnp.float32),
        pltpu.VMEM((_CHUNK * _D,), jnp.float32),
        pltpu.SemaphoreType.DMA((2,)),
        pltpu.SemaphoreType.DMA((2,)),
    ],
)


def kernel(time, table):
    idx = time.reshape(_NTOT)
    out = _gather(idx, table.reshape(_V * _D))
    return out.reshape(_BATCH, _HIST, _D)


# speedup vs baseline: 3.2878x; 3.2878x over previous
import jax
import jax.numpy as jnp
from jax import lax
from jax.experimental import pallas as pl
from jax.experimental.pallas import tpu as pltpu
from jax.experimental.pallas import tpu_sc as plsc

_D = 64
_V = 49
_BATCH = 16384
_HIST = 200
_NTOT = _BATCH * _HIST
_NC = 2
_NS = 16
_NW = _NC * _NS
_PER_W = _NTOT // _NW
_CHUNK = 800
_NCH = _PER_W // _CHUNK
_G = _CHUNK // 16


def _body(idx_hbm, table_hbm, out_hbm, table_v, idx_v0, idx_v1, rows_v0,
          rows_v1, idx_sem, out_sem):
    idx_bufs = (idx_v0, idx_v1)
    row_bufs = (rows_v0, rows_v1)
    wid = lax.axis_index("s") * _NC + lax.axis_index("c")
    base = wid * _PER_W
    pltpu.sync_copy(table_hbm, table_v)

    for b in range(2):
        off = base + b * _CHUNK
        pltpu.async_copy(idx_hbm.at[pl.ds(off, _CHUNK)], idx_bufs[b],
                         idx_sem.at[b])

    def compute(b):
        @plsc.parallel_loop(0, _G, step=1, unroll=2)
        def g_step(g):
            idx16 = idx_bufs[b][pl.ds(g * 16, 16)] * _D
            dst0 = g * (16 * _D)
            for l in range(16):
                src = idx16[l]
                dst = dst0 + l * _D
                for c in range(0, _D, 16):
                    row_bufs[b][pl.ds(dst + c, 16)] = (
                        table_v[pl.ds(src + c, 16)])

    def pair_step(i2, carry):
        for b in range(2):
            i = i2 * 2 + b
            off = base + i * _CHUNK
            ooff = off * _D
            pltpu.make_async_copy(idx_hbm.at[pl.ds(off, _CHUNK)],
                                  idx_bufs[b], idx_sem.at[b]).wait()

            @pl.when(i2 >= 1)
            def _wait_out():
                pltpu.make_async_copy(row_bufs[b],
                                      out_hbm.at[pl.ds(ooff, _CHUNK * _D)],
                                      out_sem.at[b]).wait()

            compute(b)
            pltpu.async_copy(row_bufs[b],
                             out_hbm.at[pl.ds(ooff, _CHUNK * _D)],
                             out_sem.at[b])

            @pl.when(i2 < _NCH // 2 - 1)
            def _prefetch_idx():
                off2 = base + (i + 2) * _CHUNK
                pltpu.async_copy(idx_hbm.at[pl.ds(off2, _CHUNK)],
                                 idx_bufs[b], idx_sem.at[b])
        return carry

    lax.fori_loop(0, _NCH // 2, pair_step, 0)

    for b in range(2):
        pltpu.make_async_copy(row_bufs[b],
                              out_hbm.at[pl.ds(base * _D, _CHUNK * _D)],
                              out_sem.at[b]).wait()


_mesh = plsc.VectorSubcoreMesh(core_axis_name="c", subcore_axis_name="s")

_gather = pl.kernel(
    _body,
    out_type=jax.ShapeDtypeStruct((_NTOT * _D,), jnp.float32),
    mesh=_mesh,
    compiler_params=pltpu.CompilerParams(needs_layout_passes=False),
    scratch_types=[
        pltpu.VMEM((_V * _D,), jnp.float32),
        pltpu.VMEM((_CHUNK,), jnp.int32),
        pltpu.VMEM((_CHUNK,), jnp.int32),
        pltpu.VMEM((_CHUNK * _D,), jnp.float32),
        pltpu.VMEM((_CHUNK * _D,), jnp.float32),
        pltpu.SemaphoreType.DMA((2,)),
        pltpu.SemaphoreType.DMA((2,)),
    ],
)


def kernel(time, table):
    idx = time.reshape(_NTOT)
    out = _gather(idx, table.reshape(_V * _D))
    return out.reshape(_BATCH, _HIST, _D)


# speedup vs baseline: 4.4448x; 1.3519x over previous
import jax
import jax.numpy as jnp
from jax import lax
from jax.experimental import pallas as pl
from jax.experimental.pallas import tpu as pltpu
from jax.experimental.pallas import tpu_sc as plsc

_D = 64
_V = 49
_BATCH = 16384
_HIST = 200
_NC = 2
_NS = 16
_NW = _NC * _NS
_ROWS_W = _BATCH // _NW
_RPI = 8
_RPC = 2
_NIC = _ROWS_W // _RPI
_NG = _RPC * 13


def _body(idx_hbm, table_hbm, out_hbm, table_v, idx_v0, idx_v1, rows_v0,
          rows_v1, idx_sem, out_sem):
    idx_bufs = (idx_v0, idx_v1)
    row_bufs = (rows_v0, rows_v1)
    wid = lax.axis_index("s") * _NC + lax.axis_index("c")
    base = wid * _ROWS_W
    pltpu.sync_copy(table_hbm, table_v)

    for qb in range(2):
        pltpu.async_copy(idx_hbm.at[pl.ds(base + qb * _RPI, _RPI)],
                         idx_bufs[qb], idx_sem.at[qb])

    def compute(qb, h):
        rb = h % 2

        @plsc.parallel_loop(0, _NG, step=1, unroll=2)
        def g_step(m):
            r = m // 13
            t0 = jnp.minimum((m % 13) * 16, _HIST - 16)
            idx16 = idx_bufs[qb][h * _RPC + r, pl.ds(t0, 16)] * _D
            for l in range(16):
                src = idx16[l]
                for c in range(0, _D, 16):
                    row_bufs[rb][r, t0 + l, pl.ds(c, 16)] = (
                        table_v[pl.ds(src + c, 16)])

    def q_step(q2, carry):
        for qb in range(2):
            q = q2 * 2 + qb
            pltpu.make_async_copy(
                idx_hbm.at[pl.ds(base + q * _RPI, _RPI)], idx_bufs[qb],
                idx_sem.at[qb]).wait()
            for h in range(_RPI // _RPC):
                rb = h % 2
                row0 = base + q * _RPI + h * _RPC
                dst = out_hbm.at[pl.ds(row0, _RPC)]

                def _wait_out():
                    pltpu.make_async_copy(row_bufs[rb], dst,
                                          out_sem.at[rb]).wait()

                if qb == 0 and h < 2:
                    pl.when(q2 >= 1)(_wait_out)
                else:
                    _wait_out()

                compute(qb, h)
                pltpu.async_copy(row_bufs[rb], dst, out_sem.at[rb])

            @pl.when(q2 < _NIC // 2 - 1)
            def _prefetch_idx():
                pltpu.async_copy(
                    idx_hbm.at[pl.ds(base + (q + 2) * _RPI, _RPI)],
                    idx_bufs[qb], idx_sem.at[qb])
        return carry

    lax.fori_loop(0, _NIC // 2, q_step, 0)

    for h in range(2):
        pltpu.make_async_copy(row_bufs[h], out_hbm.at[pl.ds(base, _RPC)],
                              out_sem.at[h]).wait()


_mesh = plsc.VectorSubcoreMesh(core_axis_name="c", subcore_axis_name="s")

_gather = pl.kernel(
    _body,
    out_type=jax.ShapeDtypeStruct((_BATCH, _HIST, _D), jnp.float32),
    mesh=_mesh,
    compiler_params=pltpu.CompilerParams(needs_layout_passes=False),
    scratch_types=[
        pltpu.VMEM((_V * _D,), jnp.float32),
        pltpu.VMEM((_RPI, _HIST), jnp.int32),
        pltpu.VMEM((_RPI, _HIST), jnp.int32),
        pltpu.VMEM((_RPC, _HIST, _D), jnp.float32),
        pltpu.VMEM((_RPC, _HIST, _D), jnp.float32),
        pltpu.SemaphoreType.DMA((2,)),
        pltpu.SemaphoreType.DMA((2,)),
    ],
)


def kernel(time, table):
    return _gather(time, table.reshape(_V * _D))
